# baseline (speedup 1.0000x reference)
import functools

import jax
import jax.numpy as jnp
import numpy as np
from jax import lax
from jax.experimental import pallas as pl
from jax.experimental.pallas import tpu as pltpu
from jax.experimental.pallas import tpu_sc as plsc

Q, D, CAP, A, K_NN, H = 1024, 128, 100000, 8, 32, 64
CAP_PAD = 102400
CHUNK = 2048
NSTEP = CAP_PAD // CHUNK

NC, NS, L = 2, 16, 16
NW = NC * NS
QPW = Q // NW
NB = CAP_PAD // L
SCHUNK = 10240
NCHUNK = CAP_PAD // SCHUNK
BPC = SCHUNK // L
BPG = BPC // 128
SURV_CAP = 1024
SURV_BUF = SURV_CAP + 32

_INF = np.float32(np.inf)
_NINF = np.float32(-np.inf)
_BIG = np.int32(1 << 30)


def _mlp_body(obs_ref, w1_ref, b1_ref, w2_ref, b2_ref, w3_ref, b3_ref, out_ref):
    h = jax.nn.relu(
        lax.dot_general(obs_ref[...], w1_ref[...], (((1,), (1,)), ((), ())),
                        preferred_element_type=jnp.float32)
        + b1_ref[...][None, :])
    h = jax.nn.relu(
        lax.dot_general(h, w2_ref[...], (((1,), (1,)), ((), ())),
                        preferred_element_type=jnp.float32)
        + b2_ref[...][None, :])
    out_ref[...] = (
        lax.dot_general(h, w3_ref[...], (((1,), (1,)), ((), ())),
                        preferred_element_type=jnp.float32)
        + b3_ref[...][None, :])


def _mlp(observation, W1, b1, W2, b2, W3, b3):
    return pl.pallas_call(
        _mlp_body,
        out_shape=jax.ShapeDtypeStruct((Q, A), jnp.float32),
    )(observation, W1, b1, W2, b2, W3, b3)


def _dist_body(obs_ref, obs_sq_ref, keys_ref, key_sq_ref,
               dists_ref, m_ref, dmin_ref, *, qs):
    i = pl.program_id(0)
    dot = lax.dot_general(obs_ref[...], keys_ref[...], (((1,), (1,)), ((), ())),
                          preferred_element_type=jnp.float32)
    d = (obs_sq_ref[...] - 2.0 * dot) + key_sq_ref[...][None, :]
    dists_ref[...] = d
    x = d
    w = CHUNK // 2
    while w >= 128:
        x = jnp.minimum(x[:, :w], x[:, w:2 * w])
        w //= 2
    dmin = x
    dmin_ref[...] = dmin

    @pl.when(i == 0)
    def _():
        m_ref[...] = dmin

    @pl.when(i > 0)
    def _():
        m_ref[...] = jnp.minimum(m_ref[...], dmin)


def _dists(observation, obs_sq, keys_p, key_sq_p, qs):
    return pl.pallas_call(
        functools.partial(_dist_body, qs=qs),
        grid=(NSTEP,),
        in_specs=[
            pl.BlockSpec((qs, D), lambda i: (0, 0)),
            pl.BlockSpec((qs, 1), lambda i: (0, 0)),
            pl.BlockSpec((CHUNK, D), lambda i: (i, 0)),
            pl.BlockSpec((CHUNK,), lambda i: (i,)),
        ],
        out_specs=[
            pl.BlockSpec((qs, CHUNK), lambda i: (0, i)),
            pl.BlockSpec((qs, 128), lambda i: (0, 0)),
            pl.BlockSpec((qs, 128), lambda i: (0, i)),
        ],
        out_shape=[
            jax.ShapeDtypeStruct((qs, CAP_PAD), jnp.float32),
            jax.ShapeDtypeStruct((qs, 128), jnp.float32),
            jax.ShapeDtypeStruct((qs, NB), jnp.float32),
        ],
    )(observation, obs_sq, keys_p, key_sq_p)


def _sc_body(dists_hbm, dmin_hbm, m_hbm, qnet_hbm, vals_hbm, out_hbm,
             bufa, bufb, dmina, dminb, mall_v, bidx, sdist, sidx,
             selidx_v, sel8_v, vgath_v, qnet_v, act_v, sem_a, sem_b,
             sem_da, sem_db, sem_g, *, qs, qpw):
    wid = lax.axis_index("s") * NC + lax.axis_index("c")
    q0 = wid * qpw
    lane = lax.iota(jnp.int32, L)
    pltpu.sync_copy(qnet_hbm.at[pl.ds(q0 * 16, qpw * 16)], qnet_v)
    pltpu.sync_copy(m_hbm.at[pl.ds(q0 * 128, qpw * 128)], mall_v)
    pltpu.async_copy(dmin_hbm.at[pl.ds(q0 * NB, NB)], dmina, sem_da)
    pltpu.async_copy(dists_hbm.at[q0, pl.ds(0, SCHUNK)], bufa, sem_a)

    def one_query(qq, dminbuf):
        q = q0 + qq
        t = _NINF
        for g in range(8):
            m16 = mall_v[pl.ds(qq * 128 + g * L, L)]
            tg = _NINF
            for _r in range(4):
                tg = jnp.min(m16)
                m16 = jnp.where(m16 == tg, _INF, m16)
            t = jnp.maximum(t, tg)

        bufs = (bufa, bufb)
        sems = (sem_a, sem_b)
        off = jnp.int32(0)
        for c in range(NCHUNK):
            if c + 1 < NCHUNK:
                pltpu.async_copy(
                    dists_hbm.at[q, pl.ds((c + 1) * SCHUNK, SCHUNK)],
                    bufs[(c + 1) % 2], sems[(c + 1) % 2])
            else:
                qn = jnp.minimum(q + 1, jnp.int32(qs - 1))
                pltpu.async_copy(
                    dists_hbm.at[qn, pl.ds(0, SCHUNK)], bufs[0], sems[0])
            pltpu.make_async_copy(
                dists_hbm.at[q, pl.ds(c * SCHUNK, SCHUNK)],
                bufs[c % 2], sems[c % 2]).wait()
            buf = bufs[c % 2]

            @pl.loop(0, BPG, init_carry=jnp.int32(0))
            def nb(grp, nb):
                base = c * BPC + grp * 128
                masks = []
                anym = None
                for v in range(8):
                    bm = dminbuf[pl.ds(base + v * L, L)]
                    mk = bm <= t
                    masks.append(mk)
                    anym = mk if anym is None else (anym | mk)

                def slow(nb):
                    for v in range(8):
                        mk = masks[v]
                        cum = plsc.cumsum(mk.astype(jnp.int32))
                        pos = nb + cum - 1
                        gid = lane + (grp * 128 + v * L)
                        plsc.store_scatter(bidx, [pos], gid, mask=mk)
                        nb = jnp.minimum(
                            nb + plsc.all_reduce_population_count(mk)[0],
                            jnp.int32(BPC))
                    return nb

                nhit = plsc.all_reduce_population_count(anym)
                return lax.cond(nhit[0] > 0, slow, lambda o: o, nb)

            @pl.loop(0, nb, init_carry=off)
            def off(j, off):
                b16 = bidx[pl.ds((j >> 4) << 4, L)]
                bid = jnp.max(jnp.where(lane == (j & 15), b16, jnp.int32(-1)))
                cb = (bid >> 7) * 2048 + (bid & 127)
                idx16 = cb + lane * 128
                d16 = plsc.load_gather(buf, [idx16])
                mk = d16 <= t
                cum = plsc.cumsum(mk.astype(jnp.int32))
                pos = off + cum - 1
                plsc.store_scatter(sdist, [pos], d16, mask=mk)
                plsc.store_scatter(sidx, [pos], idx16 + c * SCHUNK, mask=mk)
                return jnp.minimum(
                    off + plsc.all_reduce_population_count(mk)[0],
                    jnp.int32(SURV_CAP))

        offv = jnp.full((L,), off, jnp.int32)
        plsc.store_scatter(sdist, [offv + lane],
                           jnp.full((L,), _INF, jnp.float32))
        plsc.store_scatter(sidx, [offv + lane],
                           jnp.full((L,), _BIG, jnp.int32))
        nv = (off + 15) // 16

        @pl.loop(0, K_NN)
        def _sel(k):
            @pl.loop(0, nv, init_carry=jnp.full((L,), _INF, jnp.float32))
            def dacc(v, acc):
                return jnp.minimum(acc, sdist[pl.ds(v * L, L)])
            dmin = jnp.min(dacc)

            @pl.loop(0, nv, init_carry=jnp.full((L,), _BIG, jnp.int32))
            def iacc(v, acc):
                d16 = sdist[pl.ds(v * L, L)]
                i16 = sidx[pl.ds(v * L, L)]
                return jnp.minimum(acc, jnp.where(d16 == dmin, i16, _BIG))
            imin = jnp.min(iacc)

            @pl.loop(0, nv)
            def _mask(v):
                d16 = sdist[pl.ds(v * L, L)]
                i16 = sidx[pl.ds(v * L, L)]
                hit = (d16 == dmin) & (i16 == imin)
                sdist[pl.ds(v * L, L)] = jnp.where(hit, _INF, d16)

            plsc.store_scatter(selidx_v, [jnp.full((L,), k, jnp.int32)],
                               jnp.full((L,), imin, jnp.int32),
                               mask=lane == 0)

        s16a = selidx_v[pl.ds(0, L)]
        s16b = selidx_v[pl.ds(L, L)]
        for a in range(A):
            sel8_v[pl.ds(a * 32, L)] = s16a * A + a
            sel8_v[pl.ds(a * 32 + L, L)] = s16b * A + a
        vds = [
            pltpu.async_copy(vals_hbm.at[sel8_v.at[pl.ds(h * 128, 128)]],
                             vgath_v.at[pl.ds(h * 128, 128)], sem_g)
            for h in range(2)
        ]
        for vd in vds:
            vd.wait()

        qrow = qnet_v[pl.ds(qq * 16, L)]
        qbuf = jnp.full((L,), _NINF, jnp.float32)
        for a in range(A):
            u = (vgath_v[pl.ds(a * 32, L)] + vgath_v[pl.ds(a * 32 + L, L)])
            qa = jnp.sum(u) * jnp.float32(1.0 / K_NN)
            qa = qa + jnp.max(jnp.where(lane == a, qrow, _NINF))
            qbuf = jnp.where(lane == a, qa, qbuf)
        qm = jnp.max(qbuf)
        act = jnp.min(jnp.where(qbuf == qm, lane, _BIG))
        plsc.store_scatter(act_v, [jnp.full((L,), qq, jnp.int32)],
                           jnp.full((L,), act, jnp.int32), mask=lane == 0)

    @pl.loop(0, qpw // 2)
    def _pair(i):
        qe = q0 + 2 * i
        pltpu.async_copy(
            dmin_hbm.at[pl.ds((qe + 1) * NB, NB)], dminb, sem_db)
        pltpu.make_async_copy(
            dmin_hbm.at[pl.ds(qe * NB, NB)], dmina, sem_da).wait()
        one_query(2 * i, dmina)
        qn = jnp.minimum(qe + 2, jnp.int32(qs - 1))
        pltpu.async_copy(dmin_hbm.at[pl.ds(qn * NB, NB)], dmina, sem_da)
        pltpu.make_async_copy(
            dmin_hbm.at[pl.ds((qe + 1) * NB, NB)], dminb, sem_db).wait()
        one_query(2 * i + 1, dminb)

    pltpu.make_async_copy(
        dmin_hbm.at[pl.ds(q0 * NB, NB)], dmina, sem_da).wait()
    pltpu.make_async_copy(
        dists_hbm.at[q0, pl.ds(0, SCHUNK)], bufa, sem_a).wait()
    pltpu.sync_copy(act_v.at[pl.ds(0, qpw)], out_hbm.at[pl.ds(q0, qpw)])


def _sc_select(dists, dmin_flat, m_flat, qnet16, values_flat, qs):
    qpw = qs // NW
    mesh = plsc.VectorSubcoreMesh(core_axis_name="c", subcore_axis_name="s")
    return pl.kernel(
        functools.partial(_sc_body, qs=qs, qpw=qpw),
        out_type=jax.ShapeDtypeStruct((qs,), jnp.int32),
        mesh=mesh,
        compiler_params=pltpu.CompilerParams(needs_layout_passes=False),
        scratch_types=[
            pltpu.VMEM((SCHUNK,), jnp.float32),
            pltpu.VMEM((SCHUNK,), jnp.float32),
            pltpu.VMEM((NB,), jnp.float32),
            pltpu.VMEM((NB,), jnp.float32),
            pltpu.VMEM((qpw * 128,), jnp.float32),
            pltpu.VMEM((BPC + 16,), jnp.int32),
            pltpu.VMEM((SURV_BUF,), jnp.float32),
            pltpu.VMEM((SURV_BUF,), jnp.int32),
            pltpu.VMEM((K_NN + 16,), jnp.int32),
            pltpu.VMEM((A * 32,), jnp.int32),
            pltpu.VMEM((A * 32,), jnp.float32),
            pltpu.VMEM((qpw * 16,), jnp.float32),
            pltpu.VMEM((qpw + 16,), jnp.int32),
            pltpu.SemaphoreType.DMA,
            pltpu.SemaphoreType.DMA,
            pltpu.SemaphoreType.DMA,
            pltpu.SemaphoreType.DMA,
            pltpu.SemaphoreType.DMA,
        ],
    )(dists, dmin_flat, m_flat, qnet16, values_flat)


NSLICE = 4
QS = Q // NSLICE


def kernel(observation, keys, values, W1, b1, W2, b2, W3, b3):
    obs_sq = jnp.sum(observation * observation, axis=-1, keepdims=True)
    key_sq = jnp.sum(keys * keys, axis=-1)
    keys_p = jnp.pad(keys, ((0, CAP_PAD - CAP), (0, 0)))
    key_sq_p = jnp.pad(key_sq, (0, CAP_PAD - CAP), constant_values=np.inf)
    q_net_q = _mlp(observation, W1, b1, W2, b2, W3, b3)
    vflat = values.reshape(-1)
    acts = []
    for s in range(NSLICE):
        sl = slice(s * QS, (s + 1) * QS)
        dists, m, dmin16 = _dists(observation[sl], obs_sq[sl],
                                  keys_p, key_sq_p, QS)
        qnet16 = jnp.concatenate(
            [q_net_q[sl], q_net_q[sl]], axis=1).reshape(-1)
        acts.append(_sc_select(dists, dmin16.reshape(-1), m.reshape(-1),
                               qnet16, vflat, QS))
    return jnp.concatenate(acts)


# speedup vs baseline: 1.1596x; 1.1596x over previous
import functools

import jax
import jax.numpy as jnp
import numpy as np
from jax import lax
from jax.experimental import pallas as pl
from jax.experimental.pallas import tpu as pltpu
from jax.experimental.pallas import tpu_sc as plsc

Q, D, CAP, A, K_NN, H = 1024, 128, 100000, 8, 32, 64
CAP_PAD = 102400
CHUNK = 2048
NSTEP = CAP_PAD // CHUNK

NC, NS, L = 2, 16, 16
NW = NC * NS
QPW = Q // NW
NB = CAP_PAD // L
SCHUNK = 10240
NCHUNK = CAP_PAD // SCHUNK
BPC = SCHUNK // L
BPG = BPC // 128
NBUF = 5
PFD = 4
SURV_CAP = 1024
SURV_BUF = SURV_CAP + 32

_INF = np.float32(np.inf)
_NINF = np.float32(-np.inf)
_BIG = np.int32(1 << 30)


def _mlp_body(obs_ref, w1_ref, b1_ref, w2_ref, b2_ref, w3_ref, b3_ref, out_ref):
    h = jax.nn.relu(
        lax.dot_general(obs_ref[...], w1_ref[...], (((1,), (1,)), ((), ())),
                        preferred_element_type=jnp.float32)
        + b1_ref[...][None, :])
    h = jax.nn.relu(
        lax.dot_general(h, w2_ref[...], (((1,), (1,)), ((), ())),
                        preferred_element_type=jnp.float32)
        + b2_ref[...][None, :])
    out_ref[...] = (
        lax.dot_general(h, w3_ref[...], (((1,), (1,)), ((), ())),
                        preferred_element_type=jnp.float32)
        + b3_ref[...][None, :])


def _mlp(observation, W1, b1, W2, b2, W3, b3):
    return pl.pallas_call(
        _mlp_body,
        out_shape=jax.ShapeDtypeStruct((Q, A), jnp.float32),
    )(observation, W1, b1, W2, b2, W3, b3)


def _dist_body(obs_ref, obs_sq_ref, keys_ref, key_sq_ref,
               dists_ref, m_ref, dmin_ref, *, qs):
    i = pl.program_id(0)
    dot = lax.dot_general(obs_ref[...], keys_ref[...], (((1,), (1,)), ((), ())),
                          preferred_element_type=jnp.float32)
    d = (obs_sq_ref[...] - 2.0 * dot) + key_sq_ref[...][None, :]
    dists_ref[...] = d
    x = d
    w = CHUNK // 2
    while w >= 128:
        x = jnp.minimum(x[:, :w], x[:, w:2 * w])
        w //= 2
    dmin = x
    dmin_ref[...] = dmin

    @pl.when(i == 0)
    def _():
        m_ref[...] = dmin

    @pl.when(i > 0)
    def _():
        m_ref[...] = jnp.minimum(m_ref[...], dmin)


def _dists(observation, obs_sq, keys_p, key_sq_p, qs):
    return pl.pallas_call(
        functools.partial(_dist_body, qs=qs),
        grid=(NSTEP,),
        in_specs=[
            pl.BlockSpec((qs, D), lambda i: (0, 0)),
            pl.BlockSpec((qs, 1), lambda i: (0, 0)),
            pl.BlockSpec((CHUNK, D), lambda i: (i, 0)),
            pl.BlockSpec((CHUNK,), lambda i: (i,)),
        ],
        out_specs=[
            pl.BlockSpec((qs, CHUNK), lambda i: (0, i)),
            pl.BlockSpec((qs, 128), lambda i: (0, 0)),
            pl.BlockSpec((qs, 128), lambda i: (0, i)),
        ],
        out_shape=[
            jax.ShapeDtypeStruct((qs, CAP_PAD), jnp.float32),
            jax.ShapeDtypeStruct((qs, 128), jnp.float32),
            jax.ShapeDtypeStruct((qs, NB), jnp.float32),
        ],
    )(observation, obs_sq, keys_p, key_sq_p)


def _sc_body(dists_hbm, dmin_hbm, m_hbm, qnet_hbm, vals_hbm, out_hbm,
             buf0, buf1, buf2, buf3, buf4, dmina, dminb, mall_v, bidx,
             sdist, sidx, selidx_v, sel8_v, vgath_v, qnet_v, act_v,
             sem0, sem1, sem2, sem3, sem4,
             sem_da, sem_db, sem_g, *, qs, qpw):
    wid = lax.axis_index("s") * NC + lax.axis_index("c")
    q0 = wid * qpw
    lane = lax.iota(jnp.int32, L)
    pltpu.sync_copy(qnet_hbm.at[pl.ds(q0 * 16, qpw * 16)], qnet_v)
    pltpu.sync_copy(m_hbm.at[pl.ds(q0 * 128, qpw * 128)], mall_v)
    bufs = (buf0, buf1, buf2, buf3, buf4)
    sems = (sem0, sem1, sem2, sem3, sem4)
    pltpu.async_copy(dmin_hbm.at[pl.ds(q0 * NB, NB)], dmina, sem_da)
    for j in range(PFD):
        pltpu.async_copy(
            dists_hbm.at[q0, pl.ds(j * SCHUNK, SCHUNK)], bufs[j], sems[j])

    def one_query(qq, dminbuf):
        q = q0 + qq
        t = _NINF
        for g in range(8):
            m16 = mall_v[pl.ds(qq * 128 + g * L, L)]
            tg = _NINF
            for _r in range(4):
                tg = jnp.min(m16)
                m16 = jnp.where(m16 == tg, _INF, m16)
            t = jnp.maximum(t, tg)

        off = jnp.int32(0)
        for c in range(NCHUNK):
            tc = c + PFD
            if tc < NCHUNK:
                pltpu.async_copy(
                    dists_hbm.at[q, pl.ds(tc * SCHUNK, SCHUNK)],
                    bufs[tc % NBUF], sems[tc % NBUF])
            else:
                qn = jnp.minimum(q + 1, jnp.int32(qs - 1))
                pltpu.async_copy(
                    dists_hbm.at[qn, pl.ds((tc - NCHUNK) * SCHUNK, SCHUNK)],
                    bufs[tc % NBUF], sems[tc % NBUF])
            pltpu.make_async_copy(
                dists_hbm.at[q, pl.ds(c * SCHUNK, SCHUNK)],
                bufs[c % NBUF], sems[c % NBUF]).wait()
            buf = bufs[c % NBUF]

            @pl.loop(0, BPG, init_carry=jnp.int32(0))
            def nb(grp, nb):
                base = c * BPC + grp * 128
                masks = []
                anym = None
                for v in range(8):
                    bm = dminbuf[pl.ds(base + v * L, L)]
                    mk = bm <= t
                    masks.append(mk)
                    anym = mk if anym is None else (anym | mk)

                def slow(nb):
                    for v in range(8):
                        mk = masks[v]
                        cum = plsc.cumsum(mk.astype(jnp.int32))
                        pos = nb + cum - 1
                        gid = lane + (grp * 128 + v * L)
                        plsc.store_scatter(bidx, [pos], gid, mask=mk)
                        nb = jnp.minimum(
                            nb + plsc.all_reduce_population_count(mk)[0],
                            jnp.int32(BPC))
                    return nb

                nhit = plsc.all_reduce_population_count(anym)
                return lax.cond(nhit[0] > 0, slow, lambda o: o, nb)

            @pl.loop(0, nb, init_carry=off)
            def off(j, off):
                b16 = bidx[pl.ds((j >> 4) << 4, L)]
                bid = jnp.max(jnp.where(lane == (j & 15), b16, jnp.int32(-1)))
                cb = (bid >> 7) * 2048 + (bid & 127)
                idx16 = cb + lane * 128
                d16 = plsc.load_gather(buf, [idx16])
                mk = d16 <= t
                cum = plsc.cumsum(mk.astype(jnp.int32))
                pos = off + cum - 1
                plsc.store_scatter(sdist, [pos], d16, mask=mk)
                plsc.store_scatter(sidx, [pos], idx16 + c * SCHUNK, mask=mk)
                return jnp.minimum(
                    off + plsc.all_reduce_population_count(mk)[0],
                    jnp.int32(SURV_CAP))

        offv = jnp.full((L,), off, jnp.int32)
        plsc.store_scatter(sdist, [offv + lane],
                           jnp.full((L,), _INF, jnp.float32))
        plsc.store_scatter(sidx, [offv + lane],
                           jnp.full((L,), _BIG, jnp.int32))
        nv = (off + 15) // 16

        @pl.loop(0, K_NN)
        def _sel(k):
            @pl.loop(0, nv, init_carry=jnp.full((L,), _INF, jnp.float32))
            def dacc(v, acc):
                return jnp.minimum(acc, sdist[pl.ds(v * L, L)])
            dmin = jnp.min(dacc)

            @pl.loop(0, nv, init_carry=jnp.full((L,), _BIG, jnp.int32))
            def iacc(v, acc):
                d16 = sdist[pl.ds(v * L, L)]
                i16 = sidx[pl.ds(v * L, L)]
                return jnp.minimum(acc, jnp.where(d16 == dmin, i16, _BIG))
            imin = jnp.min(iacc)

            @pl.loop(0, nv)
            def _mask(v):
                d16 = sdist[pl.ds(v * L, L)]
                i16 = sidx[pl.ds(v * L, L)]
                hit = (d16 == dmin) & (i16 == imin)
                sdist[pl.ds(v * L, L)] = jnp.where(hit, _INF, d16)

            plsc.store_scatter(selidx_v, [jnp.full((L,), k, jnp.int32)],
                               jnp.full((L,), imin, jnp.int32),
                               mask=lane == 0)

        s16a = selidx_v[pl.ds(0, L)]
        s16b = selidx_v[pl.ds(L, L)]
        for a in range(A):
            sel8_v[pl.ds(a * 32, L)] = s16a * A + a
            sel8_v[pl.ds(a * 32 + L, L)] = s16b * A + a
        vds = [
            pltpu.async_copy(vals_hbm.at[sel8_v.at[pl.ds(h * 128, 128)]],
                             vgath_v.at[pl.ds(h * 128, 128)], sem_g)
            for h in range(2)
        ]
        for vd in vds:
            vd.wait()

        qrow = qnet_v[pl.ds(qq * 16, L)]
        qbuf = jnp.full((L,), _NINF, jnp.float32)
        for a in range(A):
            u = (vgath_v[pl.ds(a * 32, L)] + vgath_v[pl.ds(a * 32 + L, L)])
            qa = jnp.sum(u) * jnp.float32(1.0 / K_NN)
            qa = qa + jnp.max(jnp.where(lane == a, qrow, _NINF))
            qbuf = jnp.where(lane == a, qa, qbuf)
        qm = jnp.max(qbuf)
        act = jnp.min(jnp.where(qbuf == qm, lane, _BIG))
        plsc.store_scatter(act_v, [jnp.full((L,), qq, jnp.int32)],
                           jnp.full((L,), act, jnp.int32), mask=lane == 0)

    @pl.loop(0, qpw // 2)
    def _pair(i):
        qe = q0 + 2 * i
        pltpu.async_copy(
            dmin_hbm.at[pl.ds((qe + 1) * NB, NB)], dminb, sem_db)
        pltpu.make_async_copy(
            dmin_hbm.at[pl.ds(qe * NB, NB)], dmina, sem_da).wait()
        one_query(2 * i, dmina)
        qn = jnp.minimum(qe + 2, jnp.int32(qs - 1))
        pltpu.async_copy(dmin_hbm.at[pl.ds(qn * NB, NB)], dmina, sem_da)
        pltpu.make_async_copy(
            dmin_hbm.at[pl.ds((qe + 1) * NB, NB)], dminb, sem_db).wait()
        one_query(2 * i + 1, dminb)

    pltpu.make_async_copy(
        dmin_hbm.at[pl.ds(q0 * NB, NB)], dmina, sem_da).wait()
    for j in range(PFD):
        pltpu.make_async_copy(
            dists_hbm.at[q0, pl.ds(j * SCHUNK, SCHUNK)],
            bufs[j], sems[j]).wait()
    pltpu.sync_copy(act_v.at[pl.ds(0, qpw)], out_hbm.at[pl.ds(q0, qpw)])


def _sc_select(dists, dmin_flat, m_flat, qnet16, values_flat, qs):
    qpw = qs // NW
    mesh = plsc.VectorSubcoreMesh(core_axis_name="c", subcore_axis_name="s")
    return pl.kernel(
        functools.partial(_sc_body, qs=qs, qpw=qpw),
        out_type=jax.ShapeDtypeStruct((qs,), jnp.int32),
        mesh=mesh,
        compiler_params=pltpu.CompilerParams(needs_layout_passes=False),
        scratch_types=[
            pltpu.VMEM((SCHUNK,), jnp.float32),
            pltpu.VMEM((SCHUNK,), jnp.float32),
            pltpu.VMEM((SCHUNK,), jnp.float32),
            pltpu.VMEM((SCHUNK,), jnp.float32),
            pltpu.VMEM((SCHUNK,), jnp.float32),
            pltpu.VMEM((NB,), jnp.float32),
            pltpu.VMEM((NB,), jnp.float32),
            pltpu.VMEM((qpw * 128,), jnp.float32),
            pltpu.VMEM((BPC + 16,), jnp.int32),
            pltpu.VMEM((SURV_BUF,), jnp.float32),
            pltpu.VMEM((SURV_BUF,), jnp.int32),
            pltpu.VMEM((K_NN + 16,), jnp.int32),
            pltpu.VMEM((A * 32,), jnp.int32),
            pltpu.VMEM((A * 32,), jnp.float32),
            pltpu.VMEM((qpw * 16,), jnp.float32),
            pltpu.VMEM((qpw + 16,), jnp.int32),
            pltpu.SemaphoreType.DMA,
            pltpu.SemaphoreType.DMA,
            pltpu.SemaphoreType.DMA,
            pltpu.SemaphoreType.DMA,
            pltpu.SemaphoreType.DMA,
            pltpu.SemaphoreType.DMA,
            pltpu.SemaphoreType.DMA,
            pltpu.SemaphoreType.DMA,
        ],
    )(dists, dmin_flat, m_flat, qnet16, values_flat)


NSLICE = 2
QS = Q // NSLICE


def kernel(observation, keys, values, W1, b1, W2, b2, W3, b3):
    obs_sq = jnp.sum(observation * observation, axis=-1, keepdims=True)
    key_sq = jnp.sum(keys * keys, axis=-1)
    keys_p = jnp.pad(keys, ((0, CAP_PAD - CAP), (0, 0)))
    key_sq_p = jnp.pad(key_sq, (0, CAP_PAD - CAP), constant_values=np.inf)
    q_net_q = _mlp(observation, W1, b1, W2, b2, W3, b3)
    vflat = values.reshape(-1)
    acts = []
    for s in range(NSLICE):
        sl = slice(s * QS, (s + 1) * QS)
        dists, m, dmin16 = _dists(observation[sl], obs_sq[sl],
                                  keys_p, key_sq_p, QS)
        qnet16 = jnp.concatenate(
            [q_net_q[sl], q_net_q[sl]], axis=1).reshape(-1)
        acts.append(_sc_select(dists, dmin16.reshape(-1), m.reshape(-1),
                               qnet16, vflat, QS))
    return jnp.concatenate(acts)
